# baseline (speedup 1.0000x reference)
import functools

import jax
import jax.numpy as jnp
from jax import lax
from jax.experimental import pallas as pl
from jax.experimental.pallas import tpu as pltpu
from jax.experimental.pallas import tpu_sc as plsc

_D = 64
_B = 16384
_GAMMA = 12.0
_NC = 2
_NS = 16
_NW = _NC * _NS
_BPW = _B // _NW
_IC = 128
_NCHUNK = _BPW // _IC
_L = 16
_GPC = _IC // _L


def _tec_body(sample, ent, rel, out, slab, ix, h_v, t_v, cs_v, o_v, *sems):
    cid = lax.axis_index("c")
    sid = lax.axis_index("s")
    wid = sid * _NC + cid
    base = wid * _BPW

    pltpu.sync_copy(sample.at[pl.ds(base, _BPW)], slab)

    iota = lax.iota(jnp.int32, _L)

    def deint(g, carry):
        goff = g % _GPC * _L
        grow = g // _GPC
        srows = iota + g * _L
        for col in range(3):
            v = plsc.load_gather(slab, [srows, jnp.full((_L,), col, jnp.int32)])
            plsc.store_scatter(
                ix, [jnp.full((_L,), col * _NCHUNK, jnp.int32) + grow,
                     iota + goff], v)
        return carry

    lax.fori_loop(0, _BPW // _L, deint, 0)

    h_copies, t_copies, r_copies = [], [], []
    for j in range(_NCHUNK):
        rows = pl.ds(j * _IC, _IC)
        h_copies.append(pltpu.async_copy(
            ent.at[ix.at[j]], h_v.at[rows], sems[3 * j]))
        t_copies.append(pltpu.async_copy(
            ent.at[ix.at[2 * _NCHUNK + j]], t_v.at[rows], sems[3 * j + 2]))
    for j in range(_NCHUNK):
        rows = pl.ds(j * _IC, _IC)
        h_copies[j].wait()
        r_copies.append(pltpu.async_copy(
            rel.at[ix.at[_NCHUNK + j]], h_v.at[rows],
            sems[3 * j + 1], add=True))
    for j in range(_NCHUNK):
        r_copies[j].wait()
        t_copies[j].wait()

    def group(g, carry):
        def sample4(q, carry2):
            for k4 in range(4):
                k = q * 4 + k4
                s = g * _L + k
                acc = None
                for c in range(_D // _L):
                    cols = pl.ds(c * _L, _L)
                    a = jnp.abs(h_v[s, cols] - t_v[s, cols])
                    acc = a if acc is None else acc + a
                plsc.store_scatter(cs_v, [iota, iota + k], acc)
            return carry2

        lax.fori_loop(0, _L // 4, sample4, 0)
        sums = None
        for j in range(_L):
            rowv = cs_v[j, pl.ds(j, _L)]
            sums = rowv if sums is None else sums + rowv
        o_v[pl.ds(g * _L, _L)] = _GAMMA - sums
        return carry

    lax.fori_loop(0, _BPW // _L, group, 0)

    pltpu.sync_copy(o_v, out.at[pl.ds(base, _BPW)])


@functools.cache
def _build():
    mesh = plsc.VectorSubcoreMesh(
        core_axis_name="c", subcore_axis_name="s",
        num_cores=_NC, num_subcores=_NS)
    return pl.kernel(
        _tec_body,
        out_type=jax.ShapeDtypeStruct((_B,), jnp.float32),
        mesh=mesh,
        compiler_params=pltpu.CompilerParams(
            needs_layout_passes=False, use_tc_tiling_on_sc=False),
        scratch_types=[
            pltpu.VMEM((_BPW, 3), jnp.int32),
            pltpu.VMEM((3 * _NCHUNK, _IC), jnp.int32),
            pltpu.VMEM((_BPW, _D), jnp.float32),
            pltpu.VMEM((_BPW, _D), jnp.float32),
            pltpu.VMEM((_L, 2 * _L), jnp.float32),
            pltpu.VMEM((_BPW,), jnp.float32),
        ] + [pltpu.SemaphoreType.DMA] * (3 * _NCHUNK),
    )


@jax.jit
def kernel(sample, entity_embedding, relation_embedding):
    out = _build()(sample.astype(jnp.int32), entity_embedding,
                   relation_embedding)
    return out.reshape(_B, 1)


# speedup vs baseline: 1.4363x; 1.4363x over previous
import functools

import jax
import jax.numpy as jnp
from jax import lax
from jax.experimental import pallas as pl
from jax.experimental.pallas import tpu as pltpu
from jax.experimental.pallas import tpu_sc as plsc

_D = 64
_B = 16384
_GAMMA = 12.0
_NC = 2
_NS = 16
_NW = _NC * _NS
_BPW = _B // _NW
_IC = 128
_NCHUNK = _BPW // _IC
_L = 16


def _tec_body(idx_hbm, ent, rel, out, ix, h_v, t_v, cs_v, o_v, *sems):
    wid = lax.axis_index("s") * _NC + lax.axis_index("c")
    base = wid * _BPW

    pltpu.sync_copy(idx_hbm.at[wid], ix)

    def fire_h(j):
        return pltpu.async_copy(
            ent.at[ix.at[0, j]], h_v.at[pl.ds(j * _IC, _IC)], sems[3 * j])

    def fire_t(j):
        return pltpu.async_copy(
            ent.at[ix.at[2, j]], t_v.at[pl.ds(j * _IC, _IC)], sems[3 * j + 2])

    def fire_r(j):
        return pltpu.async_copy(
            rel.at[ix.at[1, j]], h_v.at[pl.ds(j * _IC, _IC)],
            sems[3 * j + 1], add=True)

    h_copies = [fire_h(0), None, None, None]
    t_copies = [fire_t(0), None, None, None]
    r_copies = [None] * _NCHUNK
    h_copies[0].wait()
    r_copies[0] = fire_r(0)
    if _NCHUNK > 1:
        h_copies[1] = fire_h(1)
        t_copies[1] = fire_t(1)

    row_ids = lax.iota(jnp.int32, _L)

    def group(g):
        for k in range(_L):
            s = g * _L + k
            acc = None
            for c in range(_D // _L):
                cols = pl.ds(c * _L, _L)
                a = jnp.abs(h_v[s, cols] - t_v[s, cols])
                acc = a if acc is None else acc + a
            plsc.store_scatter(cs_v, [row_ids, row_ids + k], acc)
        sums = None
        for j in range(_L):
            rowv = cs_v[j, pl.ds(j, _L)]
            sums = rowv if sums is None else sums + rowv
        o_v[pl.ds(g * _L, _L)] = _GAMMA - sums

    gpc = _IC // _L
    for j in range(_NCHUNK):
        if j + 1 < _NCHUNK:
            h_copies[j + 1].wait()
            r_copies[j + 1] = fire_r(j + 1)
        if j + 2 < _NCHUNK:
            h_copies[j + 2] = fire_h(j + 2)
            t_copies[j + 2] = fire_t(j + 2)
        r_copies[j].wait()
        t_copies[j].wait()

        def body(i, carry):
            group(j * gpc + i)
            return carry

        lax.fori_loop(0, gpc, body, 0, unroll=2)

    pltpu.sync_copy(o_v, out.at[pl.ds(base, _BPW)])


@functools.cache
def _build():
    mesh = plsc.VectorSubcoreMesh(
        core_axis_name="c", subcore_axis_name="s",
        num_cores=_NC, num_subcores=_NS)
    return pl.kernel(
        _tec_body,
        out_type=jax.ShapeDtypeStruct((_B,), jnp.float32),
        mesh=mesh,
        compiler_params=pltpu.CompilerParams(
            needs_layout_passes=False, use_tc_tiling_on_sc=False),
        scratch_types=[
            pltpu.VMEM((3, _NCHUNK, _IC), jnp.int32),
            pltpu.VMEM((_BPW, _D), jnp.float32),
            pltpu.VMEM((_BPW, _D), jnp.float32),
            pltpu.VMEM((_L, 2 * _L), jnp.float32),
            pltpu.VMEM((_BPW,), jnp.float32),
        ] + [pltpu.SemaphoreType.DMA] * (3 * _NCHUNK),
    )


@jax.jit
def kernel(sample, entity_embedding, relation_embedding):
    sample = sample.astype(jnp.int32)
    idx = sample.T.reshape(3, _NW, _NCHUNK, _IC).transpose(1, 0, 2, 3)
    out = _build()(idx, entity_embedding, relation_embedding)
    return out.reshape(_B, 1)


# speedup vs baseline: 1.4397x; 1.0024x over previous
import functools

import jax
import jax.numpy as jnp
from jax import lax
from jax.experimental import pallas as pl
from jax.experimental.pallas import tpu as pltpu
from jax.experimental.pallas import tpu_sc as plsc

_D = 64
_B = 16384
_GAMMA = 12.0
_NC = 2
_NS = 16
_NW = _NC * _NS
_BPW = _B // _NW
_IC = 128
_NCHUNK = _BPW // _IC
_L = 16


def _tec_body(idx_hbm, ent, rel, out, ix, h_v, t_v, cs_v, o_v, *sems):
    wid = lax.axis_index("s") * _NC + lax.axis_index("c")
    base = wid * _BPW

    pltpu.sync_copy(idx_hbm.at[wid], ix)

    h_copies, t_copies, r_copies = [], [], []
    for j in range(_NCHUNK):
        rows = pl.ds(j * _IC, _IC)
        h_copies.append(pltpu.async_copy(
            ent.at[ix.at[0, j]], h_v.at[rows], sems[3 * j]))
        t_copies.append(pltpu.async_copy(
            ent.at[ix.at[2, j]], t_v.at[rows], sems[3 * j + 2]))
    for j in range(_NCHUNK):
        rows = pl.ds(j * _IC, _IC)
        h_copies[j].wait()
        r_copies.append(pltpu.async_copy(
            rel.at[ix.at[1, j]], h_v.at[rows], sems[3 * j + 1], add=True))

    row_ids = lax.iota(jnp.int32, _L)

    def group(g):
        for k in range(_L):
            s = g * _L + k
            acc = None
            for c in range(_D // _L):
                cols = pl.ds(c * _L, _L)
                a = jnp.abs(h_v[s, cols] - t_v[s, cols])
                acc = a if acc is None else acc + a
            plsc.store_scatter(cs_v, [row_ids, row_ids + k], acc)
        sums = None
        for j in range(_L):
            rowv = cs_v[j, pl.ds(j, _L)]
            sums = rowv if sums is None else sums + rowv
        o_v[pl.ds(g * _L, _L)] = _GAMMA - sums

    gpc = _IC // _L
    for j in range(_NCHUNK):
        r_copies[j].wait()
        t_copies[j].wait()

        def body(i, carry):
            group(j * gpc + i)
            return carry

        lax.fori_loop(0, gpc, body, 0)

    pltpu.sync_copy(o_v, out.at[pl.ds(base, _BPW)])


@functools.cache
def _build():
    mesh = plsc.VectorSubcoreMesh(
        core_axis_name="c", subcore_axis_name="s",
        num_cores=_NC, num_subcores=_NS)
    return pl.kernel(
        _tec_body,
        out_type=jax.ShapeDtypeStruct((_B,), jnp.float32),
        mesh=mesh,
        compiler_params=pltpu.CompilerParams(
            needs_layout_passes=False, use_tc_tiling_on_sc=False),
        scratch_types=[
            pltpu.VMEM((3, _NCHUNK, _IC), jnp.int32),
            pltpu.VMEM((_BPW, _D), jnp.float32),
            pltpu.VMEM((_BPW, _D), jnp.float32),
            pltpu.VMEM((_L, 2 * _L), jnp.float32),
            pltpu.VMEM((_BPW,), jnp.float32),
        ] + [pltpu.SemaphoreType.DMA] * (3 * _NCHUNK),
    )


@jax.jit
def kernel(sample, entity_embedding, relation_embedding):
    sample = sample.astype(jnp.int32)
    idx = sample.T.reshape(3, _NW, _NCHUNK, _IC).transpose(1, 0, 2, 3)
    out = _build()(idx, entity_embedding, relation_embedding)
    return out.reshape(_B, 1)
